# baseline (speedup 1.0000x reference)
import functools
import jax
import jax.numpy as jnp
from jax.experimental import pallas as pl
from jax.experimental.pallas import tpu as pltpu
from jax.experimental.pallas import tpu_sc as plsc

_N = 10000
_E = 320000
_P = 640000
_H = 128
_G = 64
_ZV = 1800

_BE = 512
_NEB = _E // _BE


_NC = 2
_NS = 16
_SEGW = 8000
_NWIN = _E // _SEGW
_ACH = 160
_PP = _P + 512

_ECH = 80
_EPW = _E // (_NC * _NS)
_NECH = _EPW // _ECH
_NROW = _N // _NS


def _zero_vmem(zb, rows):
    def body(i, _):
        for j in range(8):
            zb[i, pl.ds(j * 16, 16)] = jnp.zeros((16,), jnp.float32)
        return 0
    jax.lax.fori_loop(0, rows, body, 0)


def _sc_stage_a_kernel(idx_hbm, seg_hbm, enc_hbm, table_hbm, bounds_hbm, out_hbm,
                       idx0, idx1, seg0, seg1, enc0, enc1, rows_v, zb,
                       bounds_vm, acc, table_sp, sem_g, sem_s, sem_e):
    c = jax.lax.axis_index("c")
    t = jax.lax.axis_index("s")
    slots = ((idx0, seg0, enc0), (idx1, seg1, enc1))

    def fetch(cb, b):
        idx_v, seg_v, enc_v = slots[b]
        pltpu.sync_copy(idx_hbm.at[pl.ds(cb, _ACH)], idx_v)
        pltpu.async_copy(table_sp.at[idx_v], rows_v.at[b], sem_g.at[b])
        pltpu.async_copy(seg_hbm.at[pl.ds(cb, _ACH)], seg_v, sem_s.at[b])
        pltpu.async_copy(enc_hbm.at[pl.ds(cb, _ACH)], enc_v, sem_e.at[b])

    def wait_ready(cb, b):
        idx_v, seg_v, enc_v = slots[b]
        pltpu.make_async_copy(table_sp.at[idx_v], rows_v.at[b],
                              sem_g.at[b]).wait()
        pltpu.make_async_copy(seg_hbm.at[pl.ds(cb, _ACH)], seg_v,
                              sem_s.at[b]).wait()
        pltpu.make_async_copy(enc_hbm.at[pl.ds(cb, _ACH)], enc_v,
                              sem_e.at[b]).wait()

    if True:
        pltpu.sync_copy(bounds_hbm, bounds_vm)
        _zero_vmem(zb, 40)
        @pl.when(t < 9)
        def _():
            pltpu.sync_copy(table_hbm.at[pl.ds(t * 200, 200)],
                            table_sp.at[pl.ds(t * 200, 200)])
        plsc.subcore_barrier()

        def window(p, _):
            k = 2 * p + c
            wb = k * _SEGW
            @pl.when(t < 10)
            def _():
                for q in range(20):
                    pltpu.sync_copy(zb, acc.at[pl.ds(t * 800 + q * 40, 40)])
            plsc.subcore_barrier()
            bv = bounds_vm[pl.ds(k, 16)]
            rs = bv[0]
            re = bv[1]
            per = (re - rs + _NS - 1) // _NS
            ts = jnp.minimum(rs + t * per, re)
            te = jnp.minimum(ts + per, re)
            ab = (ts // 8) * 8
            nch = (te - ab + _ACH - 1) // _ACH

            def body(j, b):
                idx_v, seg_v, enc_v = slots[b]
                cb = ab + j * _ACH
                wait_ready(cb, b)

                def grp(g, _):
                    b16 = g * 16
                    s = seg_v[pl.ds(b16, 16)] - jnp.full((16,), wb,
                                                         jnp.int32)
                    s = jnp.minimum(
                        jnp.maximum(s, jnp.zeros((16,), jnp.int32)),
                        jnp.full((16,), _SEGW - 1, jnp.int32))
                    seg_v[pl.ds(b16, 16)] = s
                    rowid = (jnp.full((16,), cb + b16, jnp.int32)
                             + jax.lax.iota(jnp.int32, 16))
                    ev = enc_v[pl.ds(b16, 16)]
                    ev = jnp.where(
                        (rowid >= jnp.full((16,), ts, jnp.int32))
                        & (rowid < jnp.full((16,), te, jnp.int32)),
                        ev, jnp.zeros((16,), jnp.float32))
                    for jr in range(16):
                        fv = jnp.full((16,), ev[jr], jnp.float32)
                        for jj in range(8):
                            cs = pl.ds(jj * 16, 16)
                            rows_v[b, b16 + jr, cs] = (
                                rows_v[b, b16 + jr, cs] * fv)
                    return 0
                jax.lax.fori_loop(0, _ACH // 16, grp, 0)
                pltpu.sync_copy(rows_v.at[b], acc.at[seg_v], add=True)
                @pl.when(j + 2 < nch)
                def _():
                    fetch(ab + (j + 2) * _ACH, b)

            @pl.when(nch > 0)
            def _():
                fetch(ab, 0)
            @pl.when(nch > 1)
            def _():
                fetch(ab + _ACH, 1)

            def chunk(j, _):
                @pl.when(j % 2 == 0)
                def _():
                    body(j, 0)
                @pl.when(j % 2 == 1)
                def _():
                    body(j, 1)
                return 0
            jax.lax.fori_loop(0, nch, chunk, 0)
            plsc.subcore_barrier()
            @pl.when(t < 10)
            def _():
                pltpu.sync_copy(acc.at[pl.ds(t * 800, 800)],
                                out_hbm.at[pl.ds(wb + t * 800, 800)])
            return 0
        jax.lax.fori_loop(0, _NWIN // _NC, window, 0)


def _sc_stage_a(pos_index, pos_enc, pos_batch, z_initial):
    bounds = jnp.searchsorted(
        pos_batch, jnp.arange(_NWIN + 1, dtype=jnp.int32) * _SEGW
    ).astype(jnp.int32)
    bounds = jnp.concatenate([bounds, jnp.zeros((48 - _NWIN - 1,), jnp.int32)])
    pad = _PP - _P
    idx = jnp.concatenate([pos_index, jnp.zeros((pad,), jnp.int32)])
    seg = jnp.concatenate([pos_batch, jnp.full((pad,), _E - 1, jnp.int32)])
    enc = jnp.concatenate([pos_enc, jnp.zeros((pad,), jnp.float32)])
    mesh = plsc.VectorSubcoreMesh(core_axis_name="c", subcore_axis_name="s")
    f = pl.kernel(
        _sc_stage_a_kernel, mesh=mesh,
        out_type=jax.ShapeDtypeStruct((_E, _H), jnp.float32),
        scratch_types=[
            pltpu.VMEM((_ACH,), jnp.int32),
            pltpu.VMEM((_ACH,), jnp.int32),
            pltpu.VMEM((_ACH,), jnp.int32),
            pltpu.VMEM((_ACH,), jnp.int32),
            pltpu.VMEM((_ACH,), jnp.float32),
            pltpu.VMEM((_ACH,), jnp.float32),
            pltpu.VMEM((2, _ACH, _H), jnp.float32),
            pltpu.VMEM((40, _H), jnp.float32),
            pltpu.VMEM((48,), jnp.int32),
            pltpu.VMEM_SHARED((_SEGW, _H), jnp.float32),
            pltpu.VMEM_SHARED((_ZV, _H), jnp.float32),
            pltpu.SemaphoreType.DMA((2,)),
            pltpu.SemaphoreType.DMA((2,)),
            pltpu.SemaphoreType.DMA((2,)),
        ],
    )
    return f(idx, seg, enc, z_initial, bounds)


def _sc_msg_kernel(e_hbm, src_hbm, dst_hbm, h_hbm, out_hbm,
                   src_v, dst_v, e_rows, h_rows, zb, acc,
                   sem_e, sem_g, sem_s):
    c = jax.lax.axis_index("c")
    t = jax.lax.axis_index("s")
    wid = t * _NC + c
    ebase = wid * _EPW

    def fetch(j, b):
        base = ebase + j * _ECH
        pltpu.sync_copy(src_hbm.at[pl.ds(base, _ECH)], src_v.at[b])
        pltpu.sync_copy(dst_hbm.at[pl.ds(base, _ECH)], dst_v.at[b])
        pltpu.async_copy(e_hbm.at[pl.ds(base, _ECH)], e_rows.at[b],
                         sem_e.at[b])
        pltpu.async_copy(h_hbm.at[src_v.at[b]], h_rows.at[b], sem_g.at[b])

    def wait_ready(j, b):
        base = ebase + j * _ECH
        pltpu.make_async_copy(e_hbm.at[pl.ds(base, _ECH)], e_rows.at[b],
                              sem_e.at[b]).wait()
        pltpu.make_async_copy(h_hbm.at[src_v.at[b]], h_rows.at[b],
                              sem_g.at[b]).wait()

    def compute_scatter(b):
        def row(i, _):
            for jj in range(8):
                sl = pl.ds(jj * 16, 16)

                e_rows[b, i, sl] = jnp.maximum(
                    e_rows[b, i, sl] + h_rows[b, i, sl],
                    jnp.zeros((16,), jnp.float32))
            return 0
        jax.lax.fori_loop(0, _ECH, row, 0)
        pltpu.async_copy(e_rows.at[b], acc.at[dst_v.at[b]], sem_s.at[b],
                         add=True)

    def wait_scatter(b):
        pltpu.make_async_copy(e_rows.at[b], acc.at[dst_v.at[b]],
                              sem_s.at[b]).wait()

    if True:
        _zero_vmem(zb, 40)
        @pl.when(t < 10)
        def _():
            for q in range(25):
                pltpu.sync_copy(zb, acc.at[pl.ds(t * 1000 + q * 40, 40)])
        plsc.subcore_barrier()

        fetch(0, 0)
        wait_ready(0, 0)
        fetch(1, 1)
        compute_scatter(0)

        def pair(jj, _):
            j1 = 2 * jj + 1
            wait_ready(j1, 1)
            wait_scatter(0)
            fetch(j1 + 1, 0)
            compute_scatter(1)
            j2 = 2 * jj + 2
            wait_ready(j2, 0)
            wait_scatter(1)
            @pl.when(jj < (_NECH - 3) // 2)
            def _():
                fetch(j2 + 1, 1)
            compute_scatter(0)
            return 0
        jax.lax.fori_loop(0, (_NECH - 1) // 2, pair, 0)
        wait_scatter(0)

        plsc.subcore_barrier()
        @pl.when(t < 10)
        def _():
            pltpu.sync_copy(acc.at[pl.ds(t * 1000, 1000)],
                            out_hbm.at[c, pl.ds(t * 1000, 1000)])


def _sc_conv_msg(e, src, dst, h):
    mesh = plsc.VectorSubcoreMesh(core_axis_name="c", subcore_axis_name="s")
    f = pl.kernel(
        _sc_msg_kernel, mesh=mesh,
        out_type=jax.ShapeDtypeStruct((_NC, _N, _H), jnp.float32),
        scratch_types=[
            pltpu.VMEM((2, _ECH), jnp.int32),
            pltpu.VMEM((2, _ECH), jnp.int32),
            pltpu.VMEM((2, _ECH, _H), jnp.float32),
            pltpu.VMEM((2, _ECH, _H), jnp.float32),
            pltpu.VMEM((40, _H), jnp.float32),
            pltpu.VMEM_SHARED((_N, _H), jnp.float32),
            pltpu.SemaphoreType.DMA((2,)),
            pltpu.SemaphoreType.DMA((2,)),
            pltpu.SemaphoreType.DMA((2,)),
        ],
    )
    return f(e, src, dst, h)


def _stats_body(z_ref, o_ref):
    @pl.when(pl.program_id(0) == 0)
    def _():
        o_ref[...] = jnp.zeros_like(o_ref)
    blk = z_ref[...]
    o_ref[0:1, :] += jnp.sum(blk, axis=0, keepdims=True)
    o_ref[1:2, :] += jnp.sum(blk * blk, axis=0, keepdims=True)


def _col_stats(z):
    return pl.pallas_call(
        _stats_body,
        grid=(_NEB,),
        in_specs=[pl.BlockSpec((_BE, _H), lambda i: (i, 0))],
        out_specs=pl.BlockSpec((8, _H), lambda i: (0, 0)),
        out_shape=jax.ShapeDtypeStruct((8, _H), jnp.float32),
    )(z)


def _bn_coeffs(s_ref, g, b, n):
    mean = s_ref[0:1, :] * (1.0 / n)
    var = s_ref[1:2, :] * (1.0 / n) - mean * mean
    scale = g * jax.lax.rsqrt(var + 1e-5)
    shift = b - mean * scale
    return scale, shift


def _mlp1_body(z_ref, s_ref, g_ref, b_ref, W_ref, wb_ref, t_ref, s2_ref):
    scale, shift = _bn_coeffs(s_ref, g_ref[...], b_ref[...], float(_E))
    z = jnp.maximum(z_ref[...] * scale + shift, 0.0)
    t = jnp.dot(z, W_ref[...], preferred_element_type=jnp.float32) + wb_ref[...]
    t_ref[...] = t
    @pl.when(pl.program_id(0) == 0)
    def _():
        s2_ref[...] = jnp.zeros_like(s2_ref)
    s2_ref[0:1, :] += jnp.sum(t, axis=0, keepdims=True)
    s2_ref[1:2, :] += jnp.sum(t * t, axis=0, keepdims=True)


def _mlp1(z, s, g, b, W, wb):
    full = pl.BlockSpec(lambda i: (0, 0))
    return pl.pallas_call(
        _mlp1_body,
        grid=(_NEB,),
        in_specs=[
            pl.BlockSpec((_BE, _H), lambda i: (i, 0)),
            pl.BlockSpec((8, _H), lambda i: (0, 0)),
            pl.BlockSpec((1, _H), lambda i: (0, 0)),
            pl.BlockSpec((1, _H), lambda i: (0, 0)),
            pl.BlockSpec((_H, _H), lambda i: (0, 0)),
            pl.BlockSpec((1, _H), lambda i: (0, 0)),
        ],
        out_specs=[
            pl.BlockSpec((_BE, _H), lambda i: (i, 0)),
            pl.BlockSpec((8, _H), lambda i: (0, 0)),
        ],
        out_shape=[
            jax.ShapeDtypeStruct((_E, _H), jnp.float32),
            jax.ShapeDtypeStruct((8, _H), jnp.float32),
        ],
    )(z, s, g, b, W, wb)


def _make_mlp2_body(n):
    def body(t_ref, s_ref, g_ref, b_ref, *rest):
        wrefs = rest[:2 * n]
        erefs = rest[2 * n:]
        scale, shift = _bn_coeffs(s_ref, g_ref[...], b_ref[...], float(_E))
        z = jnp.maximum(t_ref[...] * scale + shift, 0.0)
        for l in range(n):
            erefs[l][...] = (jnp.dot(z, wrefs[2 * l][...],
                                     preferred_element_type=jnp.float32)
                             + wrefs[2 * l + 1][...])
    return body


def _mlp2(t, s, g, b, wbs):
    n = len(wbs)
    wspec = pl.BlockSpec((_H, _H), lambda i: (0, 0))
    bspec = pl.BlockSpec((1, _H), lambda i: (0, 0))
    eshape = jax.ShapeDtypeStruct((_E, _H), jnp.float32)
    out = pl.pallas_call(
        _make_mlp2_body(n),
        grid=(_NEB,),
        in_specs=[
            pl.BlockSpec((_BE, _H), lambda i: (i, 0)),
            pl.BlockSpec((8, _H), lambda i: (0, 0)),
            bspec, bspec,
        ] + [wspec, bspec] * n,
        out_specs=[pl.BlockSpec((_BE, _H), lambda i: (i, 0))] * n,
        out_shape=[eshape] * n,
    )(t, s, g, b, *[a for wb in wbs for a in wb])
    return out


def _node_body(h_ref, a_ref, a2_ref, w1_ref, b1_ref, w2_ref, b2_ref, o_ref):
    h2 = h_ref[...] + a_ref[...] + a2_ref[...]
    h2 = jnp.maximum(jnp.dot(h2, w1_ref[...], preferred_element_type=jnp.float32)
                     + b1_ref[...], 0.0)
    o_ref[...] = jnp.maximum(jnp.dot(h2, w2_ref[...], preferred_element_type=jnp.float32)
                             + b2_ref[...], 0.0)


def _node_mlp(h, agg, agg2, w1, b1, w2, b2):
    return pl.pallas_call(
        _node_body,
        out_shape=jax.ShapeDtypeStruct((_N, _H), jnp.float32),
    )(h, agg, agg2, w1, b1, w2, b2)


_NP = 10240
_BP = 512
_NPB = _NP // _BP


def _pool_body(b_ref, h_ref, o_ref):
    @pl.when(pl.program_id(0) == 0)
    def _():
        o_ref[...] = jnp.zeros_like(o_ref)
    bb = b_ref[0]
    oh = (jax.lax.broadcasted_iota(jnp.int32, (_G, _BP), 0) == bb).astype(jnp.float32)
    o_ref[...] += jax.lax.dot_general(
        oh, h_ref[...], (((1,), (0,)), ((), ())),
        preferred_element_type=jnp.float32)


def _pool(batch3, hpad):
    return pl.pallas_call(
        _pool_body,
        grid=(_NPB,),
        in_specs=[
            pl.BlockSpec((1, 1, _BP), lambda i: (i, 0, 0)),
            pl.BlockSpec((_BP, _H), lambda i: (i, 0)),
        ],
        out_specs=pl.BlockSpec((_G, _H), lambda i: (0, 0)),
        out_shape=jax.ShapeDtypeStruct((_G, _H), jnp.float32),
    )(batch3, hpad)


def _final_body(g_ref, w1_ref, b1_ref, w2_ref, b2_ref, o_ref):
    g = jnp.maximum(jnp.dot(g_ref[...], w1_ref[...], preferred_element_type=jnp.float32)
                    + b1_ref[...], 0.0)
    g = jnp.dot(g, w2_ref[...], preferred_element_type=jnp.float32) + b2_ref[...]
    m = jnp.max(g, axis=1, keepdims=True)
    sh = g - m
    o_ref[...] = sh - jnp.log(jnp.sum(jnp.exp(sh), axis=1, keepdims=True))


def _final(g, w1, b1, w2, b2):
    return pl.pallas_call(
        _final_body,
        out_shape=jax.ShapeDtypeStruct((_G, _H), jnp.float32),
    )(g, w1, b1, w2, b2)


def kernel(x, edge_index, pos_index, pos_enc, pos_batch, batch, z_initial,
           zbn1_g, zbn1_b, zlin_W, zlin_b, zbn2_g, zbn2_b,
           conv0_eW, conv0_eb, conv0_W1, conv0_b1, conv0_W2, conv0_b2,
           conv1_eW, conv1_eb, conv1_W1, conv1_b1, conv1_W2, conv1_b2,
           conv2_eW, conv2_eb, conv2_W1, conv2_b1, conv2_W2, conv2_b2,
           lin1_W, lin1_b, lin2_W, lin2_b):
    f32 = jnp.float32
    row = lambda v: v.reshape(1, _H)

    z_raw = _sc_stage_a(pos_index, pos_enc, pos_batch, z_initial)

    s1 = _col_stats(z_raw)
    t, s2 = _mlp1(z_raw, s1, row(zbn1_g), row(zbn1_b), zlin_W, row(zlin_b))
    (e0,) = _mlp2(t, s2, row(zbn2_g), row(zbn2_b),
                  [(conv0_eW, row(conv0_eb))])
    e1, e2 = _mlp2(t, s2, row(zbn2_g), row(zbn2_b),
                   [(conv1_eW, row(conv1_eb)), (conv2_eW, row(conv2_eb))])

    src = edge_index[0]
    dst = edge_index[1]
    h = x
    convs = [
        (e0, conv0_W1, conv0_b1, conv0_W2, conv0_b2),
        (e1, conv1_W1, conv1_b1, conv1_W2, conv1_b2),
        (e2, conv2_W1, conv2_b1, conv2_W2, conv2_b2),
    ]
    for (e, W1, b1, W2, b2) in convs:
        aggp = _sc_conv_msg(e, src, dst, h)
        h = _node_mlp(h, aggp[0], aggp[1], W1, row(b1), W2, row(b2))

    hpad = jnp.concatenate([h, jnp.zeros((_NP - _N, _H), f32)], axis=0)
    bpad = jnp.concatenate([batch, jnp.zeros((_NP - _N,), jnp.int32)], axis=0)
    g = _pool(bpad.reshape(_NPB, 1, _BP), hpad)
    return _final(g, lin1_W, row(lin1_b), lin2_W, row(lin2_b))


# speedup vs baseline: 1.3631x; 1.3631x over previous
import functools
import jax
import jax.numpy as jnp
from jax.experimental import pallas as pl
from jax.experimental.pallas import tpu as pltpu
from jax.experimental.pallas import tpu_sc as plsc

_N = 10000
_E = 320000
_P = 640000
_H = 128
_G = 64
_ZV = 1800

_BE = 2000
_NEB = _E // _BE


_NC = 2
_NS = 16
_SEGW = 8000
_NWIN = _E // _SEGW
_ACH = 160
_PP = _P + 512

_ECH = 80
_EPW = _E // (_NC * _NS)
_NECH = _EPW // _ECH
_NROW = _N // _NS


def _zero_vmem(zb, rows):
    def body(i, _):
        for j in range(8):
            zb[i, pl.ds(j * 16, 16)] = jnp.zeros((16,), jnp.float32)
        return 0
    jax.lax.fori_loop(0, rows, body, 0)


def _sc_stage_a_kernel(idx_hbm, seg_hbm, enc_hbm, table_hbm, bounds_hbm, out_hbm,
                       idx0, idx1, seg0, seg1, enc0, enc1, rows_v, zb,
                       bounds_vm, acc, table_sp, sem_g, sem_s, sem_e):
    c = jax.lax.axis_index("c")
    t = jax.lax.axis_index("s")
    slots = ((idx0, seg0, enc0), (idx1, seg1, enc1))

    def fetch(cb, b):
        idx_v, seg_v, enc_v = slots[b]
        pltpu.sync_copy(idx_hbm.at[pl.ds(cb, _ACH)], idx_v)
        pltpu.async_copy(table_sp.at[idx_v], rows_v.at[b], sem_g.at[b])
        pltpu.async_copy(seg_hbm.at[pl.ds(cb, _ACH)], seg_v, sem_s.at[b])
        pltpu.async_copy(enc_hbm.at[pl.ds(cb, _ACH)], enc_v, sem_e.at[b])

    def wait_ready(cb, b):
        idx_v, seg_v, enc_v = slots[b]
        pltpu.make_async_copy(table_sp.at[idx_v], rows_v.at[b],
                              sem_g.at[b]).wait()
        pltpu.make_async_copy(seg_hbm.at[pl.ds(cb, _ACH)], seg_v,
                              sem_s.at[b]).wait()
        pltpu.make_async_copy(enc_hbm.at[pl.ds(cb, _ACH)], enc_v,
                              sem_e.at[b]).wait()

    if True:
        pltpu.sync_copy(bounds_hbm, bounds_vm)
        _zero_vmem(zb, 40)
        @pl.when(t < 9)
        def _():
            pltpu.sync_copy(table_hbm.at[pl.ds(t * 200, 200)],
                            table_sp.at[pl.ds(t * 200, 200)])
        plsc.subcore_barrier()

        def window(p, _):
            k = 2 * p + c
            wb = k * _SEGW
            @pl.when(t < 10)
            def _():
                for q in range(20):
                    pltpu.sync_copy(zb, acc.at[pl.ds(t * 800 + q * 40, 40)])
            plsc.subcore_barrier()
            bv = bounds_vm[pl.ds(k, 16)]
            rs = bv[0]
            re = bv[1]
            per = (re - rs + _NS - 1) // _NS
            ts = jnp.minimum(rs + t * per, re)
            te = jnp.minimum(ts + per, re)
            ab = (ts // 8) * 8
            nch = (te - ab + _ACH - 1) // _ACH

            def body(j, b):
                idx_v, seg_v, enc_v = slots[b]
                cb = ab + j * _ACH
                wait_ready(cb, b)

                def grp(g, _):
                    b16 = g * 16
                    s = seg_v[pl.ds(b16, 16)] - jnp.full((16,), wb,
                                                         jnp.int32)
                    s = jnp.minimum(
                        jnp.maximum(s, jnp.zeros((16,), jnp.int32)),
                        jnp.full((16,), _SEGW - 1, jnp.int32))
                    seg_v[pl.ds(b16, 16)] = s
                    rowid = (jnp.full((16,), cb + b16, jnp.int32)
                             + jax.lax.iota(jnp.int32, 16))
                    ev = enc_v[pl.ds(b16, 16)]
                    ev = jnp.where(
                        (rowid >= jnp.full((16,), ts, jnp.int32))
                        & (rowid < jnp.full((16,), te, jnp.int32)),
                        ev, jnp.zeros((16,), jnp.float32))
                    for jr in range(16):
                        fv = jnp.full((16,), ev[jr], jnp.float32)
                        for jj in range(8):
                            cs = pl.ds(jj * 16, 16)
                            rows_v[b, b16 + jr, cs] = (
                                rows_v[b, b16 + jr, cs] * fv)
                    return 0
                jax.lax.fori_loop(0, _ACH // 16, grp, 0)
                pltpu.sync_copy(rows_v.at[b], acc.at[seg_v], add=True)
                @pl.when(j + 2 < nch)
                def _():
                    fetch(ab + (j + 2) * _ACH, b)

            @pl.when(nch > 0)
            def _():
                fetch(ab, 0)
            @pl.when(nch > 1)
            def _():
                fetch(ab + _ACH, 1)

            def chunk(j, _):
                @pl.when(j % 2 == 0)
                def _():
                    body(j, 0)
                @pl.when(j % 2 == 1)
                def _():
                    body(j, 1)
                return 0
            jax.lax.fori_loop(0, nch, chunk, 0)
            plsc.subcore_barrier()
            @pl.when(t < 10)
            def _():
                pltpu.sync_copy(acc.at[pl.ds(t * 800, 800)],
                                out_hbm.at[pl.ds(wb + t * 800, 800)])
            return 0
        jax.lax.fori_loop(0, _NWIN // _NC, window, 0)


def _sc_stage_a(pos_index, pos_enc, pos_batch, z_initial):
    bounds = jnp.searchsorted(
        pos_batch, jnp.arange(_NWIN + 1, dtype=jnp.int32) * _SEGW
    ).astype(jnp.int32)
    bounds = jnp.concatenate([bounds, jnp.zeros((48 - _NWIN - 1,), jnp.int32)])
    pad = _PP - _P
    idx = jnp.concatenate([pos_index, jnp.zeros((pad,), jnp.int32)])
    seg = jnp.concatenate([pos_batch, jnp.full((pad,), _E - 1, jnp.int32)])
    enc = jnp.concatenate([pos_enc, jnp.zeros((pad,), jnp.float32)])
    mesh = plsc.VectorSubcoreMesh(core_axis_name="c", subcore_axis_name="s")
    f = pl.kernel(
        _sc_stage_a_kernel, mesh=mesh,
        out_type=jax.ShapeDtypeStruct((_E, _H), jnp.float32),
        scratch_types=[
            pltpu.VMEM((_ACH,), jnp.int32),
            pltpu.VMEM((_ACH,), jnp.int32),
            pltpu.VMEM((_ACH,), jnp.int32),
            pltpu.VMEM((_ACH,), jnp.int32),
            pltpu.VMEM((_ACH,), jnp.float32),
            pltpu.VMEM((_ACH,), jnp.float32),
            pltpu.VMEM((2, _ACH, _H), jnp.float32),
            pltpu.VMEM((40, _H), jnp.float32),
            pltpu.VMEM((48,), jnp.int32),
            pltpu.VMEM_SHARED((_SEGW, _H), jnp.float32),
            pltpu.VMEM_SHARED((_ZV, _H), jnp.float32),
            pltpu.SemaphoreType.DMA((2,)),
            pltpu.SemaphoreType.DMA((2,)),
            pltpu.SemaphoreType.DMA((2,)),
        ],
    )
    return f(idx, seg, enc, z_initial, bounds)


def _sc_msg_kernel(e_hbm, src_hbm, dst_hbm, h_hbm, out_hbm,
                   src_v, dst_v, e_rows, h_rows, zb, acc,
                   sem_e, sem_g, sem_s):
    c = jax.lax.axis_index("c")
    t = jax.lax.axis_index("s")
    wid = t * _NC + c
    ebase = wid * _EPW

    def fetch(j, b):
        base = ebase + j * _ECH
        pltpu.sync_copy(src_hbm.at[pl.ds(base, _ECH)], src_v.at[b])
        pltpu.sync_copy(dst_hbm.at[pl.ds(base, _ECH)], dst_v.at[b])
        pltpu.async_copy(e_hbm.at[pl.ds(base, _ECH)], e_rows.at[b],
                         sem_e.at[b])
        pltpu.async_copy(h_hbm.at[src_v.at[b]], h_rows.at[b], sem_g.at[b])

    def wait_ready(j, b):
        base = ebase + j * _ECH
        pltpu.make_async_copy(e_hbm.at[pl.ds(base, _ECH)], e_rows.at[b],
                              sem_e.at[b]).wait()
        pltpu.make_async_copy(h_hbm.at[src_v.at[b]], h_rows.at[b],
                              sem_g.at[b]).wait()

    def compute_scatter(b):
        def row(i, _):
            for jj in range(8):
                sl = pl.ds(jj * 16, 16)

                e_rows[b, i, sl] = jnp.maximum(
                    e_rows[b, i, sl] + h_rows[b, i, sl],
                    jnp.zeros((16,), jnp.float32))
            return 0
        jax.lax.fori_loop(0, _ECH, row, 0)
        pltpu.async_copy(e_rows.at[b], acc.at[dst_v.at[b]], sem_s.at[b],
                         add=True)

    def wait_scatter(b):
        pltpu.make_async_copy(e_rows.at[b], acc.at[dst_v.at[b]],
                              sem_s.at[b]).wait()

    if True:
        _zero_vmem(zb, 40)
        @pl.when(t < 10)
        def _():
            for q in range(25):
                pltpu.sync_copy(zb, acc.at[pl.ds(t * 1000 + q * 40, 40)])
        plsc.subcore_barrier()

        fetch(0, 0)
        wait_ready(0, 0)
        fetch(1, 1)
        compute_scatter(0)

        def pair(jj, _):
            j1 = 2 * jj + 1
            wait_ready(j1, 1)
            wait_scatter(0)
            fetch(j1 + 1, 0)
            compute_scatter(1)
            j2 = 2 * jj + 2
            wait_ready(j2, 0)
            wait_scatter(1)
            @pl.when(jj < (_NECH - 3) // 2)
            def _():
                fetch(j2 + 1, 1)
            compute_scatter(0)
            return 0
        jax.lax.fori_loop(0, (_NECH - 1) // 2, pair, 0)
        wait_scatter(0)

        plsc.subcore_barrier()
        @pl.when(t < 10)
        def _():
            pltpu.sync_copy(acc.at[pl.ds(t * 1000, 1000)],
                            out_hbm.at[c, pl.ds(t * 1000, 1000)])


def _sc_conv_msg(e, src, dst, h):
    mesh = plsc.VectorSubcoreMesh(core_axis_name="c", subcore_axis_name="s")
    f = pl.kernel(
        _sc_msg_kernel, mesh=mesh,
        out_type=jax.ShapeDtypeStruct((_NC, _N, _H), jnp.float32),
        scratch_types=[
            pltpu.VMEM((2, _ECH), jnp.int32),
            pltpu.VMEM((2, _ECH), jnp.int32),
            pltpu.VMEM((2, _ECH, _H), jnp.float32),
            pltpu.VMEM((2, _ECH, _H), jnp.float32),
            pltpu.VMEM((40, _H), jnp.float32),
            pltpu.VMEM_SHARED((_N, _H), jnp.float32),
            pltpu.SemaphoreType.DMA((2,)),
            pltpu.SemaphoreType.DMA((2,)),
            pltpu.SemaphoreType.DMA((2,)),
        ],
    )
    return f(e, src, dst, h)


def _stats_body(z_ref, o_ref):
    @pl.when(pl.program_id(0) == 0)
    def _():
        o_ref[...] = jnp.zeros_like(o_ref)
    blk = z_ref[...]
    o_ref[0:1, :] += jnp.sum(blk, axis=0, keepdims=True)
    o_ref[1:2, :] += jnp.sum(blk * blk, axis=0, keepdims=True)


def _col_stats(z):
    return pl.pallas_call(
        _stats_body,
        grid=(_NEB,),
        in_specs=[pl.BlockSpec((_BE, _H), lambda i: (i, 0))],
        out_specs=pl.BlockSpec((8, _H), lambda i: (0, 0)),
        out_shape=jax.ShapeDtypeStruct((8, _H), jnp.float32),
    )(z)


def _bn_coeffs(s_ref, g, b, n):
    mean = s_ref[0:1, :] * (1.0 / n)
    var = s_ref[1:2, :] * (1.0 / n) - mean * mean
    scale = g * jax.lax.rsqrt(var + 1e-5)
    shift = b - mean * scale
    return scale, shift


def _mlp1_body(z_ref, s_ref, g_ref, b_ref, W_ref, wb_ref, t_ref, s2_ref):
    scale, shift = _bn_coeffs(s_ref, g_ref[...], b_ref[...], float(_E))
    z = jnp.maximum(z_ref[...] * scale + shift, 0.0)
    t = jnp.dot(z, W_ref[...], preferred_element_type=jnp.float32) + wb_ref[...]
    t_ref[...] = t
    @pl.when(pl.program_id(0) == 0)
    def _():
        s2_ref[...] = jnp.zeros_like(s2_ref)
    s2_ref[0:1, :] += jnp.sum(t, axis=0, keepdims=True)
    s2_ref[1:2, :] += jnp.sum(t * t, axis=0, keepdims=True)


def _mlp1(z, s, g, b, W, wb):
    full = pl.BlockSpec(lambda i: (0, 0))
    return pl.pallas_call(
        _mlp1_body,
        grid=(_NEB,),
        in_specs=[
            pl.BlockSpec((_BE, _H), lambda i: (i, 0)),
            pl.BlockSpec((8, _H), lambda i: (0, 0)),
            pl.BlockSpec((1, _H), lambda i: (0, 0)),
            pl.BlockSpec((1, _H), lambda i: (0, 0)),
            pl.BlockSpec((_H, _H), lambda i: (0, 0)),
            pl.BlockSpec((1, _H), lambda i: (0, 0)),
        ],
        out_specs=[
            pl.BlockSpec((_BE, _H), lambda i: (i, 0)),
            pl.BlockSpec((8, _H), lambda i: (0, 0)),
        ],
        out_shape=[
            jax.ShapeDtypeStruct((_E, _H), jnp.float32),
            jax.ShapeDtypeStruct((8, _H), jnp.float32),
        ],
    )(z, s, g, b, W, wb)


def _make_mlp2_body(n):
    def body(t_ref, s_ref, g_ref, b_ref, *rest):
        wrefs = rest[:2 * n]
        erefs = rest[2 * n:]
        scale, shift = _bn_coeffs(s_ref, g_ref[...], b_ref[...], float(_E))
        z = jnp.maximum(t_ref[...] * scale + shift, 0.0)
        for l in range(n):
            erefs[l][...] = (jnp.dot(z, wrefs[2 * l][...],
                                     preferred_element_type=jnp.float32)
                             + wrefs[2 * l + 1][...])
    return body


def _mlp2(t, s, g, b, wbs):
    n = len(wbs)
    wspec = pl.BlockSpec((_H, _H), lambda i: (0, 0))
    bspec = pl.BlockSpec((1, _H), lambda i: (0, 0))
    eshape = jax.ShapeDtypeStruct((_E, _H), jnp.float32)
    out = pl.pallas_call(
        _make_mlp2_body(n),
        grid=(_NEB,),
        in_specs=[
            pl.BlockSpec((_BE, _H), lambda i: (i, 0)),
            pl.BlockSpec((8, _H), lambda i: (0, 0)),
            bspec, bspec,
        ] + [wspec, bspec] * n,
        out_specs=[pl.BlockSpec((_BE, _H), lambda i: (i, 0))] * n,
        out_shape=[eshape] * n,
    )(t, s, g, b, *[a for wb in wbs for a in wb])
    return out


def _node_body(h_ref, a_ref, a2_ref, w1_ref, b1_ref, w2_ref, b2_ref, o_ref):
    h2 = h_ref[...] + a_ref[...] + a2_ref[...]
    h2 = jnp.maximum(jnp.dot(h2, w1_ref[...], preferred_element_type=jnp.float32)
                     + b1_ref[...], 0.0)
    o_ref[...] = jnp.maximum(jnp.dot(h2, w2_ref[...], preferred_element_type=jnp.float32)
                             + b2_ref[...], 0.0)


def _node_mlp(h, agg, agg2, w1, b1, w2, b2):
    return pl.pallas_call(
        _node_body,
        out_shape=jax.ShapeDtypeStruct((_N, _H), jnp.float32),
    )(h, agg, agg2, w1, b1, w2, b2)


_NP = 10240
_BP = 512
_NPB = _NP // _BP


def _pool_body(b_ref, h_ref, o_ref):
    @pl.when(pl.program_id(0) == 0)
    def _():
        o_ref[...] = jnp.zeros_like(o_ref)
    bb = b_ref[0]
    oh = (jax.lax.broadcasted_iota(jnp.int32, (_G, _BP), 0) == bb).astype(jnp.float32)
    o_ref[...] += jax.lax.dot_general(
        oh, h_ref[...], (((1,), (0,)), ((), ())),
        preferred_element_type=jnp.float32)


def _pool(batch3, hpad):
    return pl.pallas_call(
        _pool_body,
        grid=(_NPB,),
        in_specs=[
            pl.BlockSpec((1, 1, _BP), lambda i: (i, 0, 0)),
            pl.BlockSpec((_BP, _H), lambda i: (i, 0)),
        ],
        out_specs=pl.BlockSpec((_G, _H), lambda i: (0, 0)),
        out_shape=jax.ShapeDtypeStruct((_G, _H), jnp.float32),
    )(batch3, hpad)


def _final_body(g_ref, w1_ref, b1_ref, w2_ref, b2_ref, o_ref):
    g = jnp.maximum(jnp.dot(g_ref[...], w1_ref[...], preferred_element_type=jnp.float32)
                    + b1_ref[...], 0.0)
    g = jnp.dot(g, w2_ref[...], preferred_element_type=jnp.float32) + b2_ref[...]
    m = jnp.max(g, axis=1, keepdims=True)
    sh = g - m
    o_ref[...] = sh - jnp.log(jnp.sum(jnp.exp(sh), axis=1, keepdims=True))


def _final(g, w1, b1, w2, b2):
    return pl.pallas_call(
        _final_body,
        out_shape=jax.ShapeDtypeStruct((_G, _H), jnp.float32),
    )(g, w1, b1, w2, b2)


def kernel(x, edge_index, pos_index, pos_enc, pos_batch, batch, z_initial,
           zbn1_g, zbn1_b, zlin_W, zlin_b, zbn2_g, zbn2_b,
           conv0_eW, conv0_eb, conv0_W1, conv0_b1, conv0_W2, conv0_b2,
           conv1_eW, conv1_eb, conv1_W1, conv1_b1, conv1_W2, conv1_b2,
           conv2_eW, conv2_eb, conv2_W1, conv2_b1, conv2_W2, conv2_b2,
           lin1_W, lin1_b, lin2_W, lin2_b):
    f32 = jnp.float32
    row = lambda v: v.reshape(1, _H)

    z_raw = _sc_stage_a(pos_index, pos_enc, pos_batch, z_initial)

    s1 = _col_stats(z_raw)
    t, s2 = _mlp1(z_raw, s1, row(zbn1_g), row(zbn1_b), zlin_W, row(zlin_b))
    e0, e1, e2 = _mlp2(t, s2, row(zbn2_g), row(zbn2_b),
                       [(conv0_eW, row(conv0_eb)), (conv1_eW, row(conv1_eb)),
                        (conv2_eW, row(conv2_eb))])

    src = edge_index[0]
    dst = edge_index[1]
    h = x
    convs = [
        (e0, conv0_W1, conv0_b1, conv0_W2, conv0_b2),
        (e1, conv1_W1, conv1_b1, conv1_W2, conv1_b2),
        (e2, conv2_W1, conv2_b1, conv2_W2, conv2_b2),
    ]
    for (e, W1, b1, W2, b2) in convs:
        aggp = _sc_conv_msg(e, src, dst, h)
        h = _node_mlp(h, aggp[0], aggp[1], W1, row(b1), W2, row(b2))

    hpad = jnp.concatenate([h, jnp.zeros((_NP - _N, _H), f32)], axis=0)
    bpad = jnp.concatenate([batch, jnp.zeros((_NP - _N,), jnp.int32)], axis=0)
    g = _pool(bpad.reshape(_NPB, 1, _BP), hpad)
    return _final(g, lin1_W, row(lin1_b), lin2_W, row(lin2_b))
